# baseline (speedup 1.0000x reference)
import functools

import jax
import jax.numpy as jnp
from jax import lax
from jax.experimental import pallas as pl
from jax.experimental.pallas import tpu as pltpu
from jax.experimental.pallas import tpu_sc as plsc

B, S, D, K, V, GMAX = 16, 2048, 128, 4, 256, 512
NL = K - 1
SP = S + 8

NC, NS, L = 2, 16, 16
CHUNK = S // NS
ZR = GMAX // NS
PAIRS = B * NL
PAIRS_PER_CORE = PAIRS // NC


def _embed_conv_proj(seq3_ref, emb_ref, wk_ref, projT_ref, out_ref, esp_ref):
    f32 = jnp.float32
    seq_col = seq3_ref[0]
    vio = lax.broadcasted_iota(jnp.int32, (S, V), 1)
    ohe = (seq_col == vio).astype(f32)
    esp_ref[pl.ds(0, S), :] = jnp.dot(
        ohe, emb_ref[...], preferred_element_type=f32)
    esp_ref[pl.ds(S, SP - S), :] = jnp.zeros((SP - S, D), f32)
    conv = esp_ref[pl.ds(0, S), :] * wk_ref[0:1, :]
    for k in range(1, K):
        conv = conv + esp_ref[pl.ds(k, S), :] * wk_ref[k:k + 1, :]
    out_ref[0] = jnp.dot(conv, projT_ref[...], preferred_element_type=f32)


def _sc_segment(es2_hbm, gidr_hbm, rep_hbm, sidx, oidx, dtile, otile,
                ones_t, cnt_t, zrow, nzbuf, acc, cntacc):
    f32 = jnp.float32
    cid = lax.axis_index("c")
    sid = lax.axis_index("s")
    base = sid * CHUNK

    def fill_ones(i, _):
        for v in range(D // L):
            ones_t[i, pl.ds(v * L, L)] = jnp.full((L,), 1.0, f32)
            zrow[i % ZR, pl.ds(v * L, L)] = jnp.zeros((L,), f32)
        return 0
    lax.fori_loop(0, CHUNK, fill_ones, 0)

    def pair_body(i, _):
        p = i * NC + cid
        b = p // NL
        pltpu.sync_copy(zrow, acc.at[pl.ds(sid * ZR, ZR)])
        pltpu.sync_copy(zrow, cntacc.at[pl.ds(sid * ZR, ZR)])
        plsc.subcore_barrier()
        pltpu.sync_copy(gidr_hbm.at[p, 0, pl.ds(base, CHUNK)], sidx)
        pltpu.sync_copy(es2_hbm.at[b, pl.ds(base, CHUNK)], dtile)
        pltpu.sync_copy(dtile, acc.at[sidx], add=True)
        pltpu.sync_copy(ones_t, cntacc.at[sidx], add=True)
        plsc.subcore_barrier()
        pltpu.sync_copy(cntacc.at[pl.ds(0, 1)], nzbuf)
        nz = nzbuf[0, pl.ds(0, L)][0].astype(jnp.int32)
        pltpu.sync_copy(acc.at[sidx], dtile)
        pltpu.sync_copy(cntacc.at[sidx], cnt_t)
        def scale_row(j, _):
            cvec = cnt_t[j, pl.ds(0, L)]
            scale = 1.0 / jnp.maximum(cvec, 1.0)
            valid = (base + j) >= nz
            scale = jnp.where(valid, scale, jnp.zeros((L,), f32))
            for v in range(D // L):
                otile[j, pl.ds(v * L, L)] = dtile[j, pl.ds(v * L, L)] * scale
            return 0
        lax.fori_loop(0, CHUNK, scale_row, 0)
        for g in range(CHUNK // L):
            q = lax.iota(jnp.int32, L) + (base + g * L)
            shift = jnp.where(q >= nz, -nz, S - nz)
            oidx[pl.ds(g * L, L)] = q + shift + p * S
        pltpu.sync_copy(otile, rep_hbm.at[oidx])
        plsc.subcore_barrier()
        return 0

    lax.fori_loop(0, PAIRS_PER_CORE, pair_body, 0)


_sc_segment_call = functools.partial(
    pl.kernel,
    mesh=plsc.VectorSubcoreMesh(core_axis_name="c", subcore_axis_name="s"),
    out_type=jax.ShapeDtypeStruct((PAIRS * S, D), jnp.float32),
    scratch_types=[
        pltpu.VMEM((CHUNK,), jnp.int32),
        pltpu.VMEM((CHUNK,), jnp.int32),
        pltpu.VMEM((CHUNK, D), jnp.float32),
        pltpu.VMEM((CHUNK, D), jnp.float32),
        pltpu.VMEM((CHUNK, D), jnp.float32),
        pltpu.VMEM((CHUNK, D), jnp.float32),
        pltpu.VMEM((ZR, D), jnp.float32),
        pltpu.VMEM((1, D), jnp.float32),
        pltpu.VMEM_SHARED((GMAX, D), jnp.float32),
        pltpu.VMEM_SHARED((GMAX, D), jnp.float32),
    ],
)(_sc_segment)


def _mix(idxT_ref, es2_ref, rep_ref, swT_ref, sb_ref, out_ref):
    f32 = jnp.float32
    neg = -jnp.finfo(f32).max
    bias = sb_ref[0, 0]
    reps = [es2_ref[0]] + [rep_ref[0, l] for l in range(NL)]
    scores = []
    for k in range(K):
        sc = jnp.dot(reps[k], swT_ref[...], preferred_element_type=f32) + bias
        mask = idxT_ref[0, :, k:k + 1] == 0
        scores.append(jnp.where(mask, neg, sc))
    m = jnp.maximum(jnp.maximum(scores[0], scores[1]),
                    jnp.maximum(scores[2], scores[3]))
    exps = [jnp.exp(sc - m) for sc in scores]
    denom = exps[0] + exps[1] + exps[2] + exps[3]
    out = (reps[0] * exps[0] + reps[1] * exps[1]
           + reps[2] * exps[2] + reps[3] * exps[3]) / denom
    out_ref[0] = out


@jax.jit
def kernel(sequence, group_id, emb, conv_w, proj_w, score_w, score_b):
    f32 = jnp.float32
    idxT = jnp.concatenate(
        [sequence[:, :, None], jnp.transpose(group_id, (0, 2, 1))], axis=2)
    seq3 = sequence[:, :, None]
    wk = conv_w[:, 0, :].T
    projT = proj_w[:, :, 0].T
    swT = score_w.T
    sb = score_b.reshape(1, 1).astype(f32)

    es2 = pl.pallas_call(
        _embed_conv_proj,
        grid=(B,),
        in_specs=[
            pl.BlockSpec((1, S, 1), lambda b: (b, 0, 0)),
            pl.BlockSpec((V, D), lambda b: (0, 0)),
            pl.BlockSpec((K, D), lambda b: (0, 0)),
            pl.BlockSpec((D, D), lambda b: (0, 0)),
        ],
        out_specs=pl.BlockSpec((1, S, D), lambda b: (b, 0, 0)),
        out_shape=jax.ShapeDtypeStruct((B, S, D), f32),
        scratch_shapes=[pltpu.VMEM((SP, D), f32)],
        compiler_params=pltpu.CompilerParams(
            dimension_semantics=("parallel",)),
    )(seq3, emb, wk, projT)

    gidr = group_id.reshape(PAIRS, 1, S)
    rep = _sc_segment_call(es2, gidr).reshape(B, NL, S, D)

    return pl.pallas_call(
        _mix,
        grid=(B,),
        in_specs=[
            pl.BlockSpec((1, S, K), lambda b: (b, 0, 0)),
            pl.BlockSpec((1, S, D), lambda b: (b, 0, 0)),
            pl.BlockSpec((1, NL, S, D), lambda b: (b, 0, 0, 0)),
            pl.BlockSpec((D, 1), lambda b: (0, 0)),
            pl.BlockSpec((1, 1), lambda b: (0, 0)),
        ],
        out_specs=pl.BlockSpec((1, S, D), lambda b: (b, 0, 0)),
        out_shape=jax.ShapeDtypeStruct((B, S, D), f32),
        compiler_params=pltpu.CompilerParams(
            dimension_semantics=("parallel",)),
    )(idxT, es2, rep, swT, sb)


# speedup vs baseline: 1.0164x; 1.0164x over previous
import functools

import jax
import jax.numpy as jnp
from jax import lax
from jax.experimental import pallas as pl
from jax.experimental.pallas import tpu as pltpu
from jax.experimental.pallas import tpu_sc as plsc

B, S, D, K, V, GMAX = 16, 2048, 128, 4, 256, 512
NL = K - 1
SP = S + 8

NC, NS, L = 2, 16, 16
CHUNK = S // NS
ZR = GMAX // NS
PAIRS = B * NL
PAIRS_PER_CORE = PAIRS // NC


def _embed_conv_proj(seq3_ref, emb_ref, wk_ref, projT_ref, out_ref, esp_ref):
    f32 = jnp.float32
    seq_col = seq3_ref[0]
    vio = lax.broadcasted_iota(jnp.int32, (S, V), 1)
    ohe = (seq_col == vio).astype(f32)
    esp_ref[pl.ds(0, S), :] = jnp.dot(
        ohe, emb_ref[...], preferred_element_type=f32)
    esp_ref[pl.ds(S, SP - S), :] = jnp.zeros((SP - S, D), f32)
    conv = esp_ref[pl.ds(0, S), :] * wk_ref[0:1, :]
    for k in range(1, K):
        conv = conv + esp_ref[pl.ds(k, S), :] * wk_ref[k:k + 1, :]
    out_ref[0] = jnp.dot(conv, projT_ref[...], preferred_element_type=f32)


def _sc_segment(es2_hbm, gidr_hbm, rep_hbm, sidx, oidx, dtile, otile,
                ones_t, adiv, cdiv, zrow, nzbuf, acc, cntacc, mean):
    f32 = jnp.float32
    cid = lax.axis_index("c")
    sid = lax.axis_index("s")
    base = sid * CHUNK

    def fill_ones(i, _):
        for v in range(D // L):
            ones_t[i, pl.ds(v * L, L)] = jnp.full((L,), 1.0, f32)
            zrow[i % ZR, pl.ds(v * L, L)] = jnp.zeros((L,), f32)
        return 0
    lax.fori_loop(0, CHUNK, fill_ones, 0)
    pltpu.sync_copy(zrow, acc.at[pl.ds(sid * ZR, ZR)])
    pltpu.sync_copy(zrow, cntacc.at[pl.ds(sid * ZR, ZR)])
    plsc.subcore_barrier()

    def pair_body(i, _):
        p = i * NC + cid
        b = p // NL
        pltpu.sync_copy(gidr_hbm.at[p, 0, pl.ds(base, CHUNK)], sidx)
        pltpu.sync_copy(es2_hbm.at[b, pl.ds(base, CHUNK)], dtile)
        pltpu.sync_copy(dtile, acc.at[sidx], add=True)
        pltpu.sync_copy(ones_t, cntacc.at[sidx], add=True)
        plsc.subcore_barrier()
        pltpu.sync_copy(cntacc.at[pl.ds(0, 1)], nzbuf)
        nz = nzbuf[0, pl.ds(0, L)][0].astype(jnp.int32)
        pltpu.sync_copy(acc.at[pl.ds(sid * ZR, ZR)], adiv)
        pltpu.sync_copy(cntacc.at[pl.ds(sid * ZR, ZR)], cdiv)
        plsc.subcore_barrier()
        def div_row(j, _):
            for v in range(D // L):
                c = jnp.maximum(cdiv[j, pl.ds(v * L, L)], 1.0)
                adiv[j, pl.ds(v * L, L)] = adiv[j, pl.ds(v * L, L)] / c
            return 0
        lax.fori_loop(0, ZR, div_row, 0)

        @pl.when(sid == 0)
        def _():
            for v in range(D // L):
                adiv[0, pl.ds(v * L, L)] = jnp.zeros((L,), f32)
        pltpu.sync_copy(adiv, mean.at[pl.ds(sid * ZR, ZR)])
        pltpu.sync_copy(zrow, acc.at[pl.ds(sid * ZR, ZR)])
        pltpu.sync_copy(zrow, cntacc.at[pl.ds(sid * ZR, ZR)])
        plsc.subcore_barrier()
        pltpu.sync_copy(mean.at[sidx], otile)
        for g in range(CHUNK // L):
            q = lax.iota(jnp.int32, L) + (base + g * L)
            shift = jnp.where(q >= nz, -nz, S - nz)
            oidx[pl.ds(g * L, L)] = q + shift + p * S
        pltpu.sync_copy(otile, rep_hbm.at[oidx])
        return 0

    lax.fori_loop(0, PAIRS_PER_CORE, pair_body, 0)


_sc_segment_call = functools.partial(
    pl.kernel,
    mesh=plsc.VectorSubcoreMesh(core_axis_name="c", subcore_axis_name="s"),
    out_type=jax.ShapeDtypeStruct((PAIRS * S, D), jnp.float32),
    scratch_types=[
        pltpu.VMEM((CHUNK,), jnp.int32),
        pltpu.VMEM((CHUNK,), jnp.int32),
        pltpu.VMEM((CHUNK, D), jnp.float32),
        pltpu.VMEM((CHUNK, D), jnp.float32),
        pltpu.VMEM((CHUNK, D), jnp.float32),
        pltpu.VMEM((ZR, D), jnp.float32),
        pltpu.VMEM((ZR, D), jnp.float32),
        pltpu.VMEM((ZR, D), jnp.float32),
        pltpu.VMEM((1, D), jnp.float32),
        pltpu.VMEM_SHARED((GMAX, D), jnp.float32),
        pltpu.VMEM_SHARED((GMAX, D), jnp.float32),
        pltpu.VMEM_SHARED((GMAX, D), jnp.float32),
    ],
)(_sc_segment)


def _mix(idxT_ref, es2_ref, rep_ref, swT_ref, sb_ref, out_ref):
    f32 = jnp.float32
    neg = -jnp.finfo(f32).max
    bias = sb_ref[0, 0]
    reps = [es2_ref[0]] + [rep_ref[0, l] for l in range(NL)]
    scores = []
    for k in range(K):
        sc = jnp.dot(reps[k], swT_ref[...], preferred_element_type=f32) + bias
        mask = idxT_ref[0, :, k:k + 1] == 0
        scores.append(jnp.where(mask, neg, sc))
    m = jnp.maximum(jnp.maximum(scores[0], scores[1]),
                    jnp.maximum(scores[2], scores[3]))
    exps = [jnp.exp(sc - m) for sc in scores]
    denom = exps[0] + exps[1] + exps[2] + exps[3]
    out = (reps[0] * exps[0] + reps[1] * exps[1]
           + reps[2] * exps[2] + reps[3] * exps[3]) / denom
    out_ref[0] = out


@jax.jit
def kernel(sequence, group_id, emb, conv_w, proj_w, score_w, score_b):
    f32 = jnp.float32
    idxT = jnp.concatenate(
        [sequence[:, :, None], jnp.transpose(group_id, (0, 2, 1))], axis=2)
    seq3 = sequence[:, :, None]
    wk = conv_w[:, 0, :].T
    projT = proj_w[:, :, 0].T
    swT = score_w.T
    sb = score_b.reshape(1, 1).astype(f32)

    es2 = pl.pallas_call(
        _embed_conv_proj,
        grid=(B,),
        in_specs=[
            pl.BlockSpec((1, S, 1), lambda b: (b, 0, 0)),
            pl.BlockSpec((V, D), lambda b: (0, 0)),
            pl.BlockSpec((K, D), lambda b: (0, 0)),
            pl.BlockSpec((D, D), lambda b: (0, 0)),
        ],
        out_specs=pl.BlockSpec((1, S, D), lambda b: (b, 0, 0)),
        out_shape=jax.ShapeDtypeStruct((B, S, D), f32),
        scratch_shapes=[pltpu.VMEM((SP, D), f32)],
        compiler_params=pltpu.CompilerParams(
            dimension_semantics=("parallel",)),
    )(seq3, emb, wk, projT)

    gidr = group_id.reshape(PAIRS, 1, S)
    rep = _sc_segment_call(es2, gidr).reshape(B, NL, S, D)

    return pl.pallas_call(
        _mix,
        grid=(B,),
        in_specs=[
            pl.BlockSpec((1, S, K), lambda b: (b, 0, 0)),
            pl.BlockSpec((1, S, D), lambda b: (b, 0, 0)),
            pl.BlockSpec((1, NL, S, D), lambda b: (b, 0, 0, 0)),
            pl.BlockSpec((D, 1), lambda b: (0, 0)),
            pl.BlockSpec((1, 1), lambda b: (0, 0)),
        ],
        out_specs=pl.BlockSpec((1, S, D), lambda b: (b, 0, 0)),
        out_shape=jax.ShapeDtypeStruct((B, S, D), f32),
        compiler_params=pltpu.CompilerParams(
            dimension_semantics=("parallel",)),
    )(idxT, es2, rep, swT, sb)


# speedup vs baseline: 1.1359x; 1.1176x over previous
import functools

import jax
import jax.numpy as jnp
from jax import lax
from jax.experimental import pallas as pl
from jax.experimental.pallas import tpu as pltpu
from jax.experimental.pallas import tpu_sc as plsc

B, S, D, K, V, GMAX = 16, 2048, 128, 4, 256, 512
NL = K - 1
SP = S + 8

NC, NS, L = 2, 16, 16
CHUNK = S // NS
ZR = GMAX // NS
PAIRS = B * NL
PAIRS_PER_CORE = PAIRS // NC


def _embed_conv_proj(seq3_ref, emb_ref, wk_ref, projT_ref, out_ref, esp_ref):
    f32 = jnp.float32
    seq_col = seq3_ref[0]
    vio = lax.broadcasted_iota(jnp.int32, (S, V), 1)
    ohe = (seq_col == vio).astype(f32)
    esp_ref[pl.ds(0, S), :] = jnp.dot(
        ohe, emb_ref[...], preferred_element_type=f32)
    esp_ref[pl.ds(S, SP - S), :] = jnp.zeros((SP - S, D), f32)
    conv = esp_ref[pl.ds(0, S), :] * wk_ref[0:1, :]
    for k in range(1, K):
        conv = conv + esp_ref[pl.ds(k, S), :] * wk_ref[k:k + 1, :]
    out_ref[0] = jnp.dot(conv, projT_ref[...], preferred_element_type=f32)


NG = 2


def _sc_segment(es2_hbm, gidr_hbm, rep_hbm,
                si0, si1, oi0, oi1, dt0, dt1, ot0, ot1, ones_t,
                ad0, ad1, cd0, cd1, zrow, nzbuf,
                a0, a1, c0, c1, m0, m1, sem):
    f32 = jnp.float32
    cid = lax.axis_index("c")
    sid = lax.axis_index("s")
    base = sid * CHUNK
    sidx, oidx = [si0, si1], [oi0, oi1]
    dtile, otb = [dt0, dt1], [ot0, ot1]
    adiv, cdiv = [ad0, ad1], [cd0, cd1]
    acc, cnt, mn = [a0, a1], [c0, c1], [m0, m1]

    def fill_ones(i, _):
        for v in range(D // L):
            ones_t[i, pl.ds(v * L, L)] = jnp.full((L,), 1.0, f32)
            zrow[i % ZR, pl.ds(v * L, L)] = jnp.zeros((L,), f32)
        return 0
    lax.fori_loop(0, CHUNK, fill_ones, 0)
    for j in range(NG):
        pltpu.sync_copy(zrow, acc[j].at[pl.ds(sid * ZR, ZR)])
        pltpu.sync_copy(zrow, cnt[j].at[pl.ds(sid * ZR, ZR)])
    plsc.subcore_barrier()

    def group_body(i, _):
        p0 = (i * NC + cid) * NG
        cps = []
        for j in range(NG):
            cps.append(pltpu.async_copy(
                gidr_hbm.at[p0 + j, 0, pl.ds(base, CHUNK)], sidx[j], sem))
            cps.append(pltpu.async_copy(
                es2_hbm.at[(p0 + j) // NL, pl.ds(base, CHUNK)],
                dtile[j], sem))
        for c in cps:
            c.wait()
        for j in range(NG):
            pltpu.sync_copy(dtile[j], acc[j].at[sidx[j]], add=True)
            pltpu.sync_copy(ones_t, cnt[j].at[sidx[j]], add=True)
        plsc.subcore_barrier()
        cps = []
        for j in range(NG):
            cps.append(pltpu.async_copy(cnt[j].at[pl.ds(0, 1)],
                                        nzbuf.at[pl.ds(j, 1)], sem))
            cps.append(pltpu.async_copy(acc[j].at[pl.ds(sid * ZR, ZR)],
                                        adiv[j], sem))
            cps.append(pltpu.async_copy(cnt[j].at[pl.ds(sid * ZR, ZR)],
                                        cdiv[j], sem))
        for c in cps:
            c.wait()
        plsc.subcore_barrier()
        for j in range(NG):
            def div_row(r, _):
                for v in range(D // L):
                    c = jnp.maximum(cdiv[j][r, pl.ds(v * L, L)], 1.0)
                    adiv[j][r, pl.ds(v * L, L)] = (
                        adiv[j][r, pl.ds(v * L, L)] / c)
                return 0
            lax.fori_loop(0, ZR, div_row, 0)

            @pl.when(sid == 0)
            def _():
                for v in range(D // L):
                    adiv[j][0, pl.ds(v * L, L)] = jnp.zeros((L,), f32)
        cps = []
        for j in range(NG):
            cps.append(pltpu.async_copy(adiv[j],
                                        mn[j].at[pl.ds(sid * ZR, ZR)], sem))
            cps.append(pltpu.async_copy(zrow,
                                        acc[j].at[pl.ds(sid * ZR, ZR)], sem))
            cps.append(pltpu.async_copy(zrow,
                                        cnt[j].at[pl.ds(sid * ZR, ZR)], sem))
        for c in cps:
            c.wait()
        plsc.subcore_barrier()
        for j in range(NG):
            nz = nzbuf[j, pl.ds(0, L)][0].astype(jnp.int32)
            for gg in range(CHUNK // L):
                q = lax.iota(jnp.int32, L) + (base + gg * L)
                shift = jnp.where(q >= nz, -nz, S - nz)
                oidx[j][pl.ds(gg * L, L)] = q + shift + (p0 + j) * S
        for j in range(NG):
            pltpu.sync_copy(mn[j].at[sidx[j]], otb[j])
            pltpu.sync_copy(otb[j], rep_hbm.at[oidx[j]])
        return 0

    lax.fori_loop(0, PAIRS // (NC * NG), group_body, 0)


_sc_segment_call = functools.partial(
    pl.kernel,
    mesh=plsc.VectorSubcoreMesh(core_axis_name="c", subcore_axis_name="s"),
    out_type=jax.ShapeDtypeStruct((PAIRS * S, D), jnp.float32),
    scratch_types=(
        [pltpu.VMEM((CHUNK,), jnp.int32)] * 4
        + [pltpu.VMEM((CHUNK, D), jnp.float32)] * 5
        + [pltpu.VMEM((ZR, D), jnp.float32)] * 5
        + [pltpu.VMEM((NG, D), jnp.float32)]
        + [pltpu.VMEM_SHARED((GMAX, D), jnp.float32)] * 6
        + [pltpu.SemaphoreType.DMA]
    ),
)(_sc_segment)


def _mix(idxT_ref, es2_ref, rep_ref, swT_ref, sb_ref, out_ref):
    f32 = jnp.float32
    neg = -jnp.finfo(f32).max
    bias = sb_ref[0, 0]
    reps = [es2_ref[0]] + [rep_ref[0, l] for l in range(NL)]
    scores = []
    for k in range(K):
        sc = jnp.dot(reps[k], swT_ref[...], preferred_element_type=f32) + bias
        mask = idxT_ref[0, :, k:k + 1] == 0
        scores.append(jnp.where(mask, neg, sc))
    m = jnp.maximum(jnp.maximum(scores[0], scores[1]),
                    jnp.maximum(scores[2], scores[3]))
    exps = [jnp.exp(sc - m) for sc in scores]
    denom = exps[0] + exps[1] + exps[2] + exps[3]
    out = (reps[0] * exps[0] + reps[1] * exps[1]
           + reps[2] * exps[2] + reps[3] * exps[3]) / denom
    out_ref[0] = out


@jax.jit
def kernel(sequence, group_id, emb, conv_w, proj_w, score_w, score_b):
    f32 = jnp.float32
    idxT = jnp.concatenate(
        [sequence[:, :, None], jnp.transpose(group_id, (0, 2, 1))], axis=2)
    seq3 = sequence[:, :, None]
    wk = conv_w[:, 0, :].T
    projT = proj_w[:, :, 0].T
    swT = score_w.T
    sb = score_b.reshape(1, 1).astype(f32)

    es2 = pl.pallas_call(
        _embed_conv_proj,
        grid=(B,),
        in_specs=[
            pl.BlockSpec((1, S, 1), lambda b: (b, 0, 0)),
            pl.BlockSpec((V, D), lambda b: (0, 0)),
            pl.BlockSpec((K, D), lambda b: (0, 0)),
            pl.BlockSpec((D, D), lambda b: (0, 0)),
        ],
        out_specs=pl.BlockSpec((1, S, D), lambda b: (b, 0, 0)),
        out_shape=jax.ShapeDtypeStruct((B, S, D), f32),
        scratch_shapes=[pltpu.VMEM((SP, D), f32)],
        compiler_params=pltpu.CompilerParams(
            dimension_semantics=("parallel",)),
    )(seq3, emb, wk, projT)

    gidr = group_id.reshape(PAIRS, 1, S)
    rep = _sc_segment_call(es2, gidr).reshape(B, NL, S, D)

    return pl.pallas_call(
        _mix,
        grid=(B,),
        in_specs=[
            pl.BlockSpec((1, S, K), lambda b: (b, 0, 0)),
            pl.BlockSpec((1, S, D), lambda b: (b, 0, 0)),
            pl.BlockSpec((1, NL, S, D), lambda b: (b, 0, 0, 0)),
            pl.BlockSpec((D, 1), lambda b: (0, 0)),
            pl.BlockSpec((1, 1), lambda b: (0, 0)),
        ],
        out_specs=pl.BlockSpec((1, S, D), lambda b: (b, 0, 0)),
        out_shape=jax.ShapeDtypeStruct((B, S, D), f32),
        compiler_params=pltpu.CompilerParams(
            dimension_semantics=("parallel",)),
    )(idxT, es2, rep, swT, sb)


# speedup vs baseline: 1.2959x; 1.1408x over previous
import functools

import jax
import jax.numpy as jnp
from jax import lax
from jax.experimental import pallas as pl
from jax.experimental.pallas import tpu as pltpu
from jax.experimental.pallas import tpu_sc as plsc

B, S, D, K, V, GMAX = 16, 2048, 128, 4, 256, 512
NL = K - 1
SP = S + 8

NC, NS, L = 2, 16, 16
CHUNK = S // NS
ZR = GMAX // NS
PAIRS = B * NL
NBH = B // 2
PAIRS_H = NBH * NL


def _embed_conv_proj(seq3_ref, emb_ref, wk_ref, projT_ref, out_ref, esp_ref):
    f32 = jnp.float32
    seq_col = seq3_ref[0]
    vio = lax.broadcasted_iota(jnp.int32, (S, V), 1)
    ohe = (seq_col == vio).astype(f32)
    esp_ref[pl.ds(0, S), :] = jnp.dot(
        ohe, emb_ref[...], preferred_element_type=f32)
    esp_ref[pl.ds(S, SP - S), :] = jnp.zeros((SP - S, D), f32)
    conv = esp_ref[pl.ds(0, S), :] * wk_ref[0:1, :]
    for k in range(1, K):
        conv = conv + esp_ref[pl.ds(k, S), :] * wk_ref[k:k + 1, :]
    out_ref[0] = jnp.dot(conv, projT_ref[...], preferred_element_type=f32)


NG = 2


def _sc_segment(es2_hbm, gidr_hbm, rep_hbm,
                si0, si1, oi0, oi1, dt0, dt1, ot0, ot1, ones_t,
                ad0, ad1, cd0, cd1, zrow, nzbuf,
                a0, a1, c0, c1, m0, m1, sem):
    f32 = jnp.float32
    cid = lax.axis_index("c")
    sid = lax.axis_index("s")
    base = sid * CHUNK
    sidx, oidx = [si0, si1], [oi0, oi1]
    dtile, otb = [dt0, dt1], [ot0, ot1]
    adiv, cdiv = [ad0, ad1], [cd0, cd1]
    acc, cnt, mn = [a0, a1], [c0, c1], [m0, m1]

    def fill_ones(i, _):
        for v in range(D // L):
            ones_t[i, pl.ds(v * L, L)] = jnp.full((L,), 1.0, f32)
            zrow[i % ZR, pl.ds(v * L, L)] = jnp.zeros((L,), f32)
        return 0
    lax.fori_loop(0, CHUNK, fill_ones, 0)
    for j in range(NG):
        pltpu.sync_copy(zrow, acc[j].at[pl.ds(sid * ZR, ZR)])
        pltpu.sync_copy(zrow, cnt[j].at[pl.ds(sid * ZR, ZR)])
    plsc.subcore_barrier()

    def group_body(i, _):
        p0 = (i * NC + cid) * NG
        cps = []
        for j in range(NG):
            cps.append(pltpu.async_copy(
                gidr_hbm.at[p0 + j, 0, pl.ds(base, CHUNK)], sidx[j], sem))
            cps.append(pltpu.async_copy(
                es2_hbm.at[(p0 + j) // NL, pl.ds(base, CHUNK)],
                dtile[j], sem))
        for c in cps:
            c.wait()
        for j in range(NG):
            pltpu.sync_copy(dtile[j], acc[j].at[sidx[j]], add=True)
            pltpu.sync_copy(ones_t, cnt[j].at[sidx[j]], add=True)
        plsc.subcore_barrier()
        cps = []
        for j in range(NG):
            cps.append(pltpu.async_copy(cnt[j].at[pl.ds(0, 1)],
                                        nzbuf.at[pl.ds(j, 1)], sem))
            cps.append(pltpu.async_copy(acc[j].at[pl.ds(sid * ZR, ZR)],
                                        adiv[j], sem))
            cps.append(pltpu.async_copy(cnt[j].at[pl.ds(sid * ZR, ZR)],
                                        cdiv[j], sem))
        for c in cps:
            c.wait()
        plsc.subcore_barrier()
        for j in range(NG):
            def div_row(r, _):
                for v in range(D // L):
                    c = jnp.maximum(cdiv[j][r, pl.ds(v * L, L)], 1.0)
                    adiv[j][r, pl.ds(v * L, L)] = (
                        adiv[j][r, pl.ds(v * L, L)] / c)
                return 0
            lax.fori_loop(0, ZR, div_row, 0)

            @pl.when(sid == 0)
            def _():
                for v in range(D // L):
                    adiv[j][0, pl.ds(v * L, L)] = jnp.zeros((L,), f32)
        cps = []
        for j in range(NG):
            cps.append(pltpu.async_copy(adiv[j],
                                        mn[j].at[pl.ds(sid * ZR, ZR)], sem))
            cps.append(pltpu.async_copy(zrow,
                                        acc[j].at[pl.ds(sid * ZR, ZR)], sem))
            cps.append(pltpu.async_copy(zrow,
                                        cnt[j].at[pl.ds(sid * ZR, ZR)], sem))
        for c in cps:
            c.wait()
        plsc.subcore_barrier()
        for j in range(NG):
            nz = nzbuf[j, pl.ds(0, L)][0].astype(jnp.int32)
            for gg in range(CHUNK // L):
                q = lax.iota(jnp.int32, L) + (base + gg * L)
                shift = jnp.where(q >= nz, -nz, S - nz)
                oidx[j][pl.ds(gg * L, L)] = q + shift + (p0 + j) * S
        for j in range(NG):
            pltpu.sync_copy(mn[j].at[sidx[j]], otb[j])
            pltpu.sync_copy(otb[j], rep_hbm.at[oidx[j]])
        return 0

    lax.fori_loop(0, PAIRS_H // (NC * NG), group_body, 0)


_sc_segment_call = functools.partial(
    pl.kernel,
    mesh=plsc.VectorSubcoreMesh(core_axis_name="c", subcore_axis_name="s"),
    out_type=jax.ShapeDtypeStruct((PAIRS_H * S, D), jnp.float32),
    scratch_types=(
        [pltpu.VMEM((CHUNK,), jnp.int32)] * 4
        + [pltpu.VMEM((CHUNK, D), jnp.float32)] * 5
        + [pltpu.VMEM((ZR, D), jnp.float32)] * 5
        + [pltpu.VMEM((NG, D), jnp.float32)]
        + [pltpu.VMEM_SHARED((GMAX, D), jnp.float32)] * 6
        + [pltpu.SemaphoreType.DMA]
    ),
)(_sc_segment)


def _mix(idxT_ref, es2_ref, rep_ref, swT_ref, sb_ref, out_ref):
    f32 = jnp.float32
    neg = -jnp.finfo(f32).max
    bias = sb_ref[0, 0]
    reps = [es2_ref[0]] + [rep_ref[0, l] for l in range(NL)]
    scores = []
    for k in range(K):
        sc = jnp.dot(reps[k], swT_ref[...], preferred_element_type=f32) + bias
        mask = idxT_ref[0, :, k:k + 1] == 0
        scores.append(jnp.where(mask, neg, sc))
    m = jnp.maximum(jnp.maximum(scores[0], scores[1]),
                    jnp.maximum(scores[2], scores[3]))
    exps = [jnp.exp(sc - m) for sc in scores]
    denom = exps[0] + exps[1] + exps[2] + exps[3]
    out = (reps[0] * exps[0] + reps[1] * exps[1]
           + reps[2] * exps[2] + reps[3] * exps[3]) / denom
    out_ref[0] = out


@jax.jit
def kernel(sequence, group_id, emb, conv_w, proj_w, score_w, score_b):
    f32 = jnp.float32
    idxT = jnp.concatenate(
        [sequence[:, :, None], jnp.transpose(group_id, (0, 2, 1))], axis=2)
    seq3 = sequence[:, :, None]
    wk = conv_w[:, 0, :].T
    projT = proj_w[:, :, 0].T
    swT = score_w.T
    sb = score_b.reshape(1, 1).astype(f32)

    def phase_a(seq3_h):
        return pl.pallas_call(
            _embed_conv_proj,
            grid=(NBH,),
            in_specs=[
                pl.BlockSpec((1, S, 1), lambda b: (b, 0, 0)),
                pl.BlockSpec((V, D), lambda b: (0, 0)),
                pl.BlockSpec((K, D), lambda b: (0, 0)),
                pl.BlockSpec((D, D), lambda b: (0, 0)),
            ],
            out_specs=pl.BlockSpec((1, S, D), lambda b: (b, 0, 0)),
            out_shape=jax.ShapeDtypeStruct((NBH, S, D), f32),
            scratch_shapes=[pltpu.VMEM((SP, D), f32)],
            compiler_params=pltpu.CompilerParams(
                dimension_semantics=("parallel",)),
        )(seq3_h, emb, wk, projT)

    def phase_c(idxT_h, es2_h, rep_h):
        return pl.pallas_call(
            _mix,
            grid=(NBH,),
            in_specs=[
                pl.BlockSpec((1, S, K), lambda b: (b, 0, 0)),
                pl.BlockSpec((1, S, D), lambda b: (b, 0, 0)),
                pl.BlockSpec((1, NL, S, D), lambda b: (b, 0, 0, 0)),
                pl.BlockSpec((D, 1), lambda b: (0, 0)),
                pl.BlockSpec((1, 1), lambda b: (0, 0)),
            ],
            out_specs=pl.BlockSpec((1, S, D), lambda b: (b, 0, 0)),
            out_shape=jax.ShapeDtypeStruct((NBH, S, D), f32),
            compiler_params=pltpu.CompilerParams(
                dimension_semantics=("parallel",)),
        )(idxT_h, es2_h, rep_h, swT, sb)

    outs = []
    es2s, reps = [], []
    for h in range(2):
        es2s.append(phase_a(seq3[h * NBH:(h + 1) * NBH]))
        gidr_h = group_id[h * NBH:(h + 1) * NBH].reshape(PAIRS_H, 1, S)
        reps.append(_sc_segment_call(es2s[h], gidr_h)
                    .reshape(NBH, NL, S, D))
    for h in range(2):
        outs.append(phase_c(idxT[h * NBH:(h + 1) * NBH], es2s[h], reps[h]))
    return jnp.concatenate(outs, axis=0)


# speedup vs baseline: 1.3872x; 1.0704x over previous
import functools

import jax
import jax.numpy as jnp
from jax import lax
from jax.experimental import pallas as pl
from jax.experimental.pallas import tpu as pltpu
from jax.experimental.pallas import tpu_sc as plsc

B, S, D, K, V, GMAX = 16, 2048, 128, 4, 256, 512
NL = K - 1
SP = S + 8

NC, NS, L = 2, 16, 16
CHUNK = S // NS
ZR = GMAX // NS
PAIRS = B * NL
NBH = B // 4
PAIRS_H = NBH * NL


def _embed_conv_proj(seq3_ref, emb_ref, wk_ref, projT_ref, out_ref, esp_ref):
    f32 = jnp.float32
    seq_col = seq3_ref[0]
    vio = lax.broadcasted_iota(jnp.int32, (S, V), 1)
    ohe = (seq_col == vio).astype(f32)
    esp_ref[pl.ds(0, S), :] = jnp.dot(
        ohe, emb_ref[...], preferred_element_type=f32)
    esp_ref[pl.ds(S, SP - S), :] = jnp.zeros((SP - S, D), f32)
    conv = esp_ref[pl.ds(0, S), :] * wk_ref[0:1, :]
    for k in range(1, K):
        conv = conv + esp_ref[pl.ds(k, S), :] * wk_ref[k:k + 1, :]
    out_ref[0] = jnp.dot(conv, projT_ref[...], preferred_element_type=f32)


NG = 2


def _sc_segment(es2_hbm, gidr_hbm, rep_hbm,
                si0, si1, oi0, oi1, dt0, dt1, ot0, ot1, ones_t,
                ad0, ad1, cd0, cd1, zrow, nzbuf,
                a0, a1, c0, c1, m0, m1, sem):
    f32 = jnp.float32
    cid = lax.axis_index("c")
    sid = lax.axis_index("s")
    base = sid * CHUNK
    sidx, oidx = [si0, si1], [oi0, oi1]
    dtile, otb = [dt0, dt1], [ot0, ot1]
    adiv, cdiv = [ad0, ad1], [cd0, cd1]
    acc, cnt, mn = [a0, a1], [c0, c1], [m0, m1]

    def fill_ones(i, _):
        for v in range(D // L):
            ones_t[i, pl.ds(v * L, L)] = jnp.full((L,), 1.0, f32)
            zrow[i % ZR, pl.ds(v * L, L)] = jnp.zeros((L,), f32)
        return 0
    lax.fori_loop(0, CHUNK, fill_ones, 0)
    for j in range(NG):
        pltpu.sync_copy(zrow, acc[j].at[pl.ds(sid * ZR, ZR)])
        pltpu.sync_copy(zrow, cnt[j].at[pl.ds(sid * ZR, ZR)])
    plsc.subcore_barrier()

    def group_body(i, _):
        p0 = (i * NC + cid) * NG
        cps = []
        for j in range(NG):
            cps.append(pltpu.async_copy(
                gidr_hbm.at[p0 + j, 0, pl.ds(base, CHUNK)], sidx[j], sem))
            cps.append(pltpu.async_copy(
                es2_hbm.at[(p0 + j) // NL, pl.ds(base, CHUNK)],
                dtile[j], sem))
        for c in cps:
            c.wait()
        for j in range(NG):
            pltpu.sync_copy(dtile[j], acc[j].at[sidx[j]], add=True)
            pltpu.sync_copy(ones_t, cnt[j].at[sidx[j]], add=True)
        plsc.subcore_barrier()
        cps = []
        for j in range(NG):
            cps.append(pltpu.async_copy(cnt[j].at[pl.ds(0, 1)],
                                        nzbuf.at[pl.ds(j, 1)], sem))
            cps.append(pltpu.async_copy(acc[j].at[pl.ds(sid * ZR, ZR)],
                                        adiv[j], sem))
            cps.append(pltpu.async_copy(cnt[j].at[pl.ds(sid * ZR, ZR)],
                                        cdiv[j], sem))
        for c in cps:
            c.wait()
        plsc.subcore_barrier()
        for j in range(NG):
            def div_row(r, _):
                for v in range(D // L):
                    c = jnp.maximum(cdiv[j][r, pl.ds(v * L, L)], 1.0)
                    adiv[j][r, pl.ds(v * L, L)] = (
                        adiv[j][r, pl.ds(v * L, L)] / c)
                return 0
            lax.fori_loop(0, ZR, div_row, 0)

            @pl.when(sid == 0)
            def _():
                for v in range(D // L):
                    adiv[j][0, pl.ds(v * L, L)] = jnp.zeros((L,), f32)
        cps = []
        for j in range(NG):
            cps.append(pltpu.async_copy(adiv[j],
                                        mn[j].at[pl.ds(sid * ZR, ZR)], sem))
            cps.append(pltpu.async_copy(zrow,
                                        acc[j].at[pl.ds(sid * ZR, ZR)], sem))
            cps.append(pltpu.async_copy(zrow,
                                        cnt[j].at[pl.ds(sid * ZR, ZR)], sem))
        for c in cps:
            c.wait()
        plsc.subcore_barrier()
        for j in range(NG):
            nz = nzbuf[j, pl.ds(0, L)][0].astype(jnp.int32)
            for gg in range(CHUNK // L):
                q = lax.iota(jnp.int32, L) + (base + gg * L)
                shift = jnp.where(q >= nz, -nz, S - nz)
                oidx[j][pl.ds(gg * L, L)] = q + shift + (p0 + j) * S
        for j in range(NG):
            pltpu.sync_copy(mn[j].at[sidx[j]], otb[j])
            pltpu.sync_copy(otb[j], rep_hbm.at[oidx[j]])
        return 0

    lax.fori_loop(0, PAIRS_H // (NC * NG), group_body, 0)


_sc_segment_call = functools.partial(
    pl.kernel,
    mesh=plsc.VectorSubcoreMesh(core_axis_name="c", subcore_axis_name="s"),
    out_type=jax.ShapeDtypeStruct((PAIRS_H * S, D), jnp.float32),
    scratch_types=(
        [pltpu.VMEM((CHUNK,), jnp.int32)] * 4
        + [pltpu.VMEM((CHUNK, D), jnp.float32)] * 5
        + [pltpu.VMEM((ZR, D), jnp.float32)] * 5
        + [pltpu.VMEM((NG, D), jnp.float32)]
        + [pltpu.VMEM_SHARED((GMAX, D), jnp.float32)] * 6
        + [pltpu.SemaphoreType.DMA]
    ),
)(_sc_segment)


def _mix(idxT_ref, es2_ref, rep_ref, swT_ref, sb_ref, out_ref):
    f32 = jnp.float32
    neg = -jnp.finfo(f32).max
    bias = sb_ref[0, 0]
    reps = [es2_ref[0]] + [rep_ref[0, l] for l in range(NL)]
    scores = []
    for k in range(K):
        sc = jnp.dot(reps[k], swT_ref[...], preferred_element_type=f32) + bias
        mask = idxT_ref[0, :, k:k + 1] == 0
        scores.append(jnp.where(mask, neg, sc))
    m = jnp.maximum(jnp.maximum(scores[0], scores[1]),
                    jnp.maximum(scores[2], scores[3]))
    exps = [jnp.exp(sc - m) for sc in scores]
    denom = exps[0] + exps[1] + exps[2] + exps[3]
    out = (reps[0] * exps[0] + reps[1] * exps[1]
           + reps[2] * exps[2] + reps[3] * exps[3]) / denom
    out_ref[0] = out


@jax.jit
def kernel(sequence, group_id, emb, conv_w, proj_w, score_w, score_b):
    f32 = jnp.float32
    idxT = jnp.concatenate(
        [sequence[:, :, None], jnp.transpose(group_id, (0, 2, 1))], axis=2)
    seq3 = sequence[:, :, None]
    wk = conv_w[:, 0, :].T
    projT = proj_w[:, :, 0].T
    swT = score_w.T
    sb = score_b.reshape(1, 1).astype(f32)

    def phase_a(seq3_h):
        return pl.pallas_call(
            _embed_conv_proj,
            grid=(NBH,),
            in_specs=[
                pl.BlockSpec((1, S, 1), lambda b: (b, 0, 0)),
                pl.BlockSpec((V, D), lambda b: (0, 0)),
                pl.BlockSpec((K, D), lambda b: (0, 0)),
                pl.BlockSpec((D, D), lambda b: (0, 0)),
            ],
            out_specs=pl.BlockSpec((1, S, D), lambda b: (b, 0, 0)),
            out_shape=jax.ShapeDtypeStruct((NBH, S, D), f32),
            scratch_shapes=[pltpu.VMEM((SP, D), f32)],
            compiler_params=pltpu.CompilerParams(
                dimension_semantics=("parallel",)),
        )(seq3_h, emb, wk, projT)

    def phase_c(idxT_h, es2_h, rep_h):
        return pl.pallas_call(
            _mix,
            grid=(NBH,),
            in_specs=[
                pl.BlockSpec((1, S, K), lambda b: (b, 0, 0)),
                pl.BlockSpec((1, S, D), lambda b: (b, 0, 0)),
                pl.BlockSpec((1, NL, S, D), lambda b: (b, 0, 0, 0)),
                pl.BlockSpec((D, 1), lambda b: (0, 0)),
                pl.BlockSpec((1, 1), lambda b: (0, 0)),
            ],
            out_specs=pl.BlockSpec((1, S, D), lambda b: (b, 0, 0)),
            out_shape=jax.ShapeDtypeStruct((NBH, S, D), f32),
            compiler_params=pltpu.CompilerParams(
                dimension_semantics=("parallel",)),
        )(idxT_h, es2_h, rep_h, swT, sb)

    outs = []
    es2s, reps = [], []
    for h in range(4):
        es2s.append(phase_a(seq3[h * NBH:(h + 1) * NBH]))
        gidr_h = group_id[h * NBH:(h + 1) * NBH].reshape(PAIRS_H, 1, S)
        reps.append(_sc_segment_call(es2s[h], gidr_h)
                    .reshape(NBH, NL, S, D))
    for h in range(4):
        outs.append(phase_c(idxT[h * NBH:(h + 1) * NBH], es2s[h], reps[h]))
    return jnp.concatenate(outs, axis=0)
